# baseline (speedup 1.0000x reference)
import jax
import jax.numpy as jnp
from jax import lax
from jax.experimental import pallas as pl
from jax.experimental.pallas import tpu as pltpu


def _vq_pair_kernel(idx_ref, wt_ref, z_ref, zq_ref, cnt_ref, err_ref):
    idx = idx_ref[...]
    k, d = wt_ref.shape
    t2 = idx.shape[1]
    t = t2 // 2

    row_iota = lax.broadcasted_iota(jnp.int32, (k, t2), 0)
    mask = row_iota == idx
    onehot_bf = mask.astype(jnp.bfloat16)

    zq = lax.dot_general(wt_ref[...], onehot_bf,
                         dimension_numbers=(((0,), (0,)), ((), ())),
                         preferred_element_type=jnp.float32)
    zq_ref[0] = zq[:, :t]
    zq_ref[1] = zq[:, t:]

    cnt_ref[...] = jnp.sum(mask.astype(jnp.float32), axis=1, keepdims=True)

    e = jnp.zeros((1, 1), jnp.float32)
    for i in range(2):
        diff = zq[:, i * t:(i + 1) * t] - z_ref[i]
        sq = diff * diff
        e += jnp.sum(jnp.sum(sq.reshape(d // 8, 8, t), axis=0)).reshape(1, 1)
    err_ref[...] = e


def kernel(encoding_indices, z, weight, cluster_size_buf):
    b, d, h, w = z.shape
    hw = h * w
    n = b * hw
    k = weight.shape[0]
    beta = 0.25
    bp = b // 2

    idx = encoding_indices.astype(jnp.int32).reshape(bp, 1, 2 * hw)
    w_t = jnp.asarray(weight, jnp.float32).astype(jnp.bfloat16)
    z_flat = z.reshape(b, d, hw)

    grid = (bp,)
    idx_spec = pl.BlockSpec((None, 1, 2 * hw), lambda bi: (bi, 0, 0))
    wt_spec = pl.BlockSpec(memory_space=pltpu.MemorySpace.VMEM)
    tok_spec = pl.BlockSpec((2, d, hw), lambda bi: (bi, 0, 0))
    cnt_spec = pl.BlockSpec((None, k, 1), lambda bi: (bi, 0, 0))
    err_spec = pl.BlockSpec((None, 1, 1), lambda bi: (bi, 0, 0))

    cparams = pltpu.CompilerParams(
        dimension_semantics=("arbitrary",),
        vmem_limit_bytes=64 << 20)

    zq_nc, cnt_part, err_part = pl.pallas_call(
        _vq_pair_kernel,
        out_shape=(
            jax.ShapeDtypeStruct((b, d, hw), jnp.float32),
            jax.ShapeDtypeStruct((bp, k, 1), jnp.float32),
            jax.ShapeDtypeStruct((bp, 1, 1), jnp.float32),
        ),
        grid_spec=pltpu.PrefetchScalarGridSpec(
            num_scalar_prefetch=0,
            grid=grid,
            in_specs=[idx_spec, wt_spec, tok_spec],
            out_specs=[tok_spec, cnt_spec, err_spec],
        ),
        compiler_params=cparams,
    )(idx, w_t, z_flat)

    z_q = zq_nc.reshape(b, d, h, w)
    loss = beta * jnp.sum(err_part) / jnp.float32(n * d)
    counts = jnp.sum(cnt_part[:, :, 0], axis=0)
    new_cluster_size = counts + 0.0 * cluster_size_buf

    return z_q, loss, encoding_indices, new_cluster_size
